# baseline (speedup 1.0000x reference)
import jax
import jax.numpy as jnp
from jax.experimental import pallas as pl

B, H, L_Q, U, L_K = 4, 16, 4096, 128, 4096

ROWS_PER_BLOCK = 512
N_ROWS = B * H * U
N_BLOCKS = N_ROWS // ROWS_PER_BLOCK
L_KW = L_K // 4

ONES = 0x01010101


def _mask_kernel(q_ref, bnd_ref, out_ref):
    w = jax.lax.broadcasted_iota(jnp.int32, out_ref.shape, 1)
    q = q_ref[...]
    boundary = jnp.where(w == q, bnd_ref[...], 0)
    out_ref[...] = jnp.where(w > q, ONES, boundary)


def kernel(index, scores):
    del scores
    t = index.reshape(N_ROWS, 1).astype(jnp.int32) + 1
    q = t >> 2
    bnd = ONES << (8 * (t & 3))
    row_spec = pl.BlockSpec((ROWS_PER_BLOCK, 1), lambda i: (i, 0))
    packed = pl.pallas_call(
        _mask_kernel,
        grid=(N_BLOCKS,),
        in_specs=[row_spec, row_spec],
        out_specs=pl.BlockSpec((ROWS_PER_BLOCK, L_KW), lambda i: (i, 0)),
        out_shape=jax.ShapeDtypeStruct((N_ROWS, L_KW), jnp.int32),
    )(q, bnd)
    out8 = jax.lax.bitcast_convert_type(packed, jnp.int8)
    return (out8 != 0).reshape(B, H, U, L_K)


# speedup vs baseline: 7.2748x; 7.2748x over previous
import jax
import jax.numpy as jnp
from jax.experimental import pallas as pl

B, H, L_Q, U, L_K = 4, 16, 4096, 128, 4096

ROWS_PER_BLOCK = 512
N_ROWS = B * H * U
N_BLOCKS = N_ROWS // ROWS_PER_BLOCK


def _mask_kernel(idx_ref, out_ref):
    col = jax.lax.broadcasted_iota(jnp.int32, out_ref.shape, 1)
    out_ref[...] = (col > idx_ref[...]).astype(jnp.int8)


def kernel(index, scores):
    del scores
    idx = index.reshape(N_ROWS, 1).astype(jnp.int32)
    out = pl.pallas_call(
        _mask_kernel,
        grid=(N_BLOCKS,),
        in_specs=[pl.BlockSpec((ROWS_PER_BLOCK, 1), lambda i: (i, 0))],
        out_specs=pl.BlockSpec((ROWS_PER_BLOCK, L_K), lambda i: (i, 0)),
        out_shape=jax.ShapeDtypeStruct((N_ROWS, L_K), jnp.int8),
    )(idx)
    return (out != 0).reshape(B, H, U, L_K)
